# baseline (speedup 1.0000x reference)
import functools

import jax
import jax.numpy as jnp
from jax import lax
from jax.experimental import pallas as pl
from jax.experimental.pallas import tpu as pltpu
from jax.experimental.pallas import tpu_sc as plsc

N_NODES_C = 100000
N_HEDGES_C = 200000
N_TYPES_C = 16
B_C = 4096
H_C = 4
S_C = 32
D_C = 128

NC = 2
NS = 16
NW = NC * NS
BPW = B_C // NW
RPW = BPW * H_C
NCHUNK = H_C
CR = BPW
EPC = CR * S_C
HT_PAD = 200704
EPT = HT_PAD // NS
WPT = EPT // 8
TPW = HT_PAD // 8


def _sc_body(npT_hbm, th_hbm, lb_hbm, nbrF_hbm, ht_hbm, emb_hbm,
             pooled_hbm, nm_hbm,
             np_v, th_v, lb_v, tp_v, raw_v, pk_v,
             ix0, ix1, ix2, ix3, nb0, nb1, nb2, nb3,
             eb0, eb1, hist_v, nm_v, tp_sh,
             sem_ne, sem_emb, sem_tp):
    wid = lax.axis_index("s") * NC + lax.axis_index("c")
    sid = lax.axis_index("s")
    base_b = wid * BPW
    iota16 = lax.iota(jnp.int32, 16)
    ones = jnp.ones((16,), jnp.float32)

    for h in range(H_C):
        pltpu.sync_copy(npT_hbm.at[pl.ds(h * B_C + base_b, BPW)],
                        np_v.at[pl.ds(h * BPW, BPW)])

    tp_cp = pltpu.async_copy(ht_hbm.at[pl.ds(sid * EPT, EPT)], raw_v, sem_tp)

    ixbufs = [ix0, ix1, ix2, ix3]
    nbufs = [nb0, nb1, nb2, nb3]
    ebufs = [eb0, eb1]
    ne_cp = [[None] * S_C for _ in range(NCHUNK)]
    emb_cp = [None] * NCHUNK
    soff = iota16 * N_NODES_C

    def _fire_ne(c):
        ix = ixbufs[c]

        @plsc.parallel_loop(0, CR)
        def _mkidx(r):
            n = plsc.load_gather(
                np_v, [jnp.full((16,), c * CR + r, jnp.int32)])
            for k in range(2):
                ix[pl.ds(r * S_C + k * 16, 16)] = n + (soff + k * 16 * N_NODES_C)

        for j in range(S_C):
            ne_cp[c][j] = pltpu.async_copy(
                nbrF_hbm.at[ix.at[pl.ds(j * 128, 128)]],
                nbufs[c].at[pl.ds(j * 128, 128)], sem_ne)

    def _fire_emb(c):
        emb_cp[c] = pltpu.async_copy(
            emb_hbm.at[np_v.at[pl.ds(c * CR, CR)]], ebufs[c % 2], sem_emb)

    _fire_emb(0)
    _fire_emb(1)
    _fire_ne(0)
    _fire_ne(1)

    pltpu.sync_copy(th_hbm.at[pl.ds(base_b, BPW)], th_v)
    pltpu.sync_copy(lb_hbm.at[pl.ds(base_b, BPW)], lb_v)

    tp_cp.wait()

    @plsc.parallel_loop(0, WPT // 16)
    def _pack(i):
        j = (jnp.full((16,), i * 16, jnp.int32) + iota16) * 8
        w = plsc.load_gather(raw_v, [j])
        for k in range(1, 8):
            w = w | lax.shift_left(plsc.load_gather(raw_v, [j + k]), 4 * k)
        pk_v[pl.ds(i * 16, 16)] = w

    pltpu.sync_copy(pk_v, tp_sh.at[pl.ds(sid * WPT, WPT)])

    @plsc.parallel_loop(0, BPW)
    def _zero(i):
        hist_v[i, pl.ds(0, 16)] = jnp.zeros((16,), jnp.float32)

    plsc.subcore_barrier()
    pltpu.sync_copy(tp_sh, tp_v)

    for c in range(NCHUNK):
        for j in range(S_C):
            ne_cp[c][j].wait()
        nb = nbufs[c]

        @plsc.parallel_loop(0, CR, unroll=4)
        def _hist(r, _nb=nb):
            rv = jnp.full((16,), r, jnp.int32)
            thv = plsc.load_gather(th_v, [rv])
            for k in range(2):
                e = _nb[pl.ds(r * S_C + k * 16, 16)]
                w = plsc.load_gather(tp_v, [lax.shift_right_logical(e, 3)])
                sh = lax.shift_left(jnp.bitwise_and(e, 7), 2)
                t = jnp.bitwise_and(lax.shift_right_logical(w, sh), 15)
                m = jnp.not_equal(e, thv)
                plsc.addupdate_scatter(hist_v, [rv, t], ones, mask=m)

        if c + 2 < NCHUNK:
            _fire_ne(c + 2)

    @plsc.parallel_loop(0, BPW)
    def _pooled(b):
        hv = hist_v[b, pl.ds(0, 16)]
        lbl = plsc.load_gather(lb_v, [jnp.full((16,), b, jnp.int32)])
        onehot = jnp.where(iota16 == lbl, 1.0, 0.0).astype(jnp.float32)
        hist_v[b, pl.ds(0, 16)] = hv * (1.0 / 128.0) + onehot

    pltpu.sync_copy(hist_v,
                    pooled_hbm.at[pl.ds(base_b, BPW), pl.ds(0, N_TYPES_C)])

    for c in range(NCHUNK):
        emb_cp[c].wait()
        eb = ebufs[c % 2]

        @plsc.parallel_loop(0, CR, unroll=4)
        def _emb(i, _c=c, _eb=eb):
            for d in range(8):
                v = _eb[i, pl.ds(d * 16, 16)]
                if _c == 0:
                    nm_v[i, pl.ds(d * 16, 16)] = v
                else:
                    nm_v[i, pl.ds(d * 16, 16)] += v

        if c + 2 < NCHUNK:
            _fire_emb(c + 2)

    pltpu.sync_copy(nm_v, nm_hbm.at[pl.ds(base_b, BPW)])


_sc_gather = functools.partial(
    pl.kernel,
    out_type=(
        jax.ShapeDtypeStruct((B_C, D_C), jnp.float32),
        jax.ShapeDtypeStruct((B_C, D_C), jnp.float32),
    ),
    mesh=plsc.VectorSubcoreMesh(core_axis_name="c", subcore_axis_name="s"),
    compiler_params=pltpu.CompilerParams(needs_layout_passes=False,
                                         use_tc_tiling_on_sc=False),
    scratch_types=[
        pltpu.VMEM((RPW,), jnp.int32),
        pltpu.VMEM((BPW,), jnp.int32),
        pltpu.VMEM((BPW,), jnp.int32),
        pltpu.VMEM((TPW,), jnp.int32),
        pltpu.VMEM((EPT,), jnp.int32),
        pltpu.VMEM((WPT,), jnp.int32),
        pltpu.VMEM((EPC,), jnp.int32),
        pltpu.VMEM((EPC,), jnp.int32),
        pltpu.VMEM((EPC,), jnp.int32),
        pltpu.VMEM((EPC,), jnp.int32),
        pltpu.VMEM((EPC,), jnp.int32),
        pltpu.VMEM((EPC,), jnp.int32),
        pltpu.VMEM((EPC,), jnp.int32),
        pltpu.VMEM((EPC,), jnp.int32),
        pltpu.VMEM((CR, D_C), jnp.float32),
        pltpu.VMEM((CR, D_C), jnp.float32),
        pltpu.VMEM((BPW, N_TYPES_C), jnp.float32),
        pltpu.VMEM((BPW, D_C), jnp.float32),
        pltpu.VMEM_SHARED((TPW,), jnp.int32),
        pltpu.SemaphoreType.DMA,
        pltpu.SemaphoreType.DMA,
        pltpu.SemaphoreType.DMA,
    ],
)(_sc_body)


def _tc_body(pooled_ref, nm_ref, w1_ref, b1_ref, w2_ref, b2_ref,
             sc_ref, v2_ref):
    v1 = jnp.dot(pooled_ref[:, :N_TYPES_C], w1_ref[...],
                 preferred_element_type=jnp.float32) + b1_ref[...]
    sc_ref[...] = jax.nn.sigmoid(v1)
    p2 = jnp.dot(nm_ref[...], w2_ref[...],
                 preferred_element_type=jnp.float32) + b2_ref[...]
    v2_ref[...] = jnp.concatenate([v1, p2], axis=1)


_tc_heads = pl.pallas_call(
    _tc_body,
    out_shape=(
        jax.ShapeDtypeStruct((B_C, N_TYPES_C), jnp.float32),
        jax.ShapeDtypeStruct((B_C, 2 * N_TYPES_C), jnp.float32),
    ),
)


@jax.jit
def kernel(node_pairs, train_hedges, labels, neighborhedges, hedgetypes,
           nodeEmb, W1, b1, W2, b2):
    npT = node_pairs.astype(jnp.int32).T.reshape(-1)
    th = train_hedges.astype(jnp.int32)
    lb = labels.astype(jnp.int32)
    nbrF = neighborhedges.astype(jnp.int32).T.reshape(-1)
    ht = jnp.concatenate([hedgetypes.astype(jnp.int32),
                          jnp.zeros((HT_PAD - N_HEDGES_C,), jnp.int32)])

    pooled, embsum = _sc_gather(npT, th, lb, nbrF, ht, nodeEmb)

    scores, vector2 = _tc_heads(pooled, embsum, W1, b1.reshape(1, -1),
                                W2 * 0.25, b2.reshape(1, -1))
    return (scores, vector2)


# speedup vs baseline: 1.0049x; 1.0049x over previous
import functools

import jax
import jax.numpy as jnp
from jax import lax
from jax.experimental import pallas as pl
from jax.experimental.pallas import tpu as pltpu
from jax.experimental.pallas import tpu_sc as plsc

N_NODES_C = 100000
N_HEDGES_C = 200000
N_TYPES_C = 16
B_C = 4096
H_C = 4
S_C = 32
D_C = 128

NC = 2
NS = 16
NW = NC * NS
BPW = B_C // NW
RPW = BPW * H_C
NCHUNK = H_C
CR = BPW
EPC = CR * S_C
HT_PAD = 200704
EPT = HT_PAD // NS
WPT = EPT // 8
TPW = HT_PAD // 8


def _sc_body(npT_hbm, th_hbm, lb_hbm, nbrF_hbm, ht_hbm, emb_hbm,
             pooled_hbm, nm_hbm,
             np_v, th_v, lb_v, tp_v, raw_v, pk_v,
             ix0, ix1, nb0, nb1,
             eb0, eb1, hist_v, nm_v, tp_sh,
             sem_ne, sem_emb, sem_tp):
    wid = lax.axis_index("s") * NC + lax.axis_index("c")
    sid = lax.axis_index("s")
    base_b = wid * BPW
    iota16 = lax.iota(jnp.int32, 16)
    ones = jnp.ones((16,), jnp.float32)

    for h in range(H_C):
        pltpu.sync_copy(npT_hbm.at[pl.ds(h * B_C + base_b, BPW)],
                        np_v.at[pl.ds(h * BPW, BPW)])

    tp_cp = pltpu.async_copy(ht_hbm.at[pl.ds(sid * EPT, EPT)], raw_v, sem_tp)

    ixbufs = [ix0, ix1]
    nbufs = [nb0, nb1]
    ebufs = [eb0, eb1]
    ne_cp = [[None] * S_C for _ in range(NCHUNK)]
    emb_cp = [None] * NCHUNK
    soff = iota16 * N_NODES_C

    def _fire_ne(c):
        ix = ixbufs[c % 2]

        @plsc.parallel_loop(0, CR)
        def _mkidx(r):
            n = plsc.load_gather(
                np_v, [jnp.full((16,), c * CR + r, jnp.int32)])
            for k in range(2):
                ix[pl.ds(r * S_C + k * 16, 16)] = n + (soff + k * 16 * N_NODES_C)

        for j in range(S_C):
            ne_cp[c][j] = pltpu.async_copy(
                nbrF_hbm.at[ix.at[pl.ds(j * 128, 128)]],
                nbufs[c % 2].at[pl.ds(j * 128, 128)], sem_ne)

    def _fire_emb(c):
        emb_cp[c] = pltpu.async_copy(
            emb_hbm.at[np_v.at[pl.ds(c * CR, CR)]], ebufs[c % 2], sem_emb)

    _fire_emb(0)
    _fire_emb(1)
    _fire_ne(0)
    _fire_ne(1)

    pltpu.sync_copy(th_hbm.at[pl.ds(base_b, BPW)], th_v)
    pltpu.sync_copy(lb_hbm.at[pl.ds(base_b, BPW)], lb_v)

    tp_cp.wait()

    @plsc.parallel_loop(0, WPT // 16)
    def _pack(i):
        j = (jnp.full((16,), i * 16, jnp.int32) + iota16) * 8
        w = plsc.load_gather(raw_v, [j])
        for k in range(1, 8):
            w = w | lax.shift_left(plsc.load_gather(raw_v, [j + k]), 4 * k)
        pk_v[pl.ds(i * 16, 16)] = w

    pltpu.sync_copy(pk_v, tp_sh.at[pl.ds(sid * WPT, WPT)])

    @plsc.parallel_loop(0, BPW)
    def _zero(i):
        hist_v[i, pl.ds(0, 16)] = jnp.zeros((16,), jnp.float32)

    plsc.subcore_barrier()
    pltpu.sync_copy(tp_sh, tp_v)

    for c in range(NCHUNK):
        for j in range(S_C):
            ne_cp[c][j].wait()
        nb = nbufs[c % 2]

        @plsc.parallel_loop(0, CR)
        def _hist(r, _nb=nb):
            rv = jnp.full((16,), r, jnp.int32)
            thv = plsc.load_gather(th_v, [rv])
            for k in range(2):
                e = _nb[pl.ds(r * S_C + k * 16, 16)]
                w = plsc.load_gather(tp_v, [lax.shift_right_logical(e, 3)])
                sh = lax.shift_left(jnp.bitwise_and(e, 7), 2)
                t = jnp.bitwise_and(lax.shift_right_logical(w, sh), 15)
                m = jnp.not_equal(e, thv)
                plsc.addupdate_scatter(hist_v, [rv, t], ones, mask=m)

        if c + 2 < NCHUNK:
            _fire_ne(c + 2)

    @plsc.parallel_loop(0, BPW)
    def _pooled(b):
        hv = hist_v[b, pl.ds(0, 16)]
        lbl = plsc.load_gather(lb_v, [jnp.full((16,), b, jnp.int32)])
        onehot = jnp.where(iota16 == lbl, 1.0, 0.0).astype(jnp.float32)
        hist_v[b, pl.ds(0, 16)] = hv * (1.0 / 128.0) + onehot

    pltpu.sync_copy(hist_v,
                    pooled_hbm.at[pl.ds(base_b, BPW), pl.ds(0, N_TYPES_C)])

    for c in range(NCHUNK):
        emb_cp[c].wait()
        eb = ebufs[c % 2]

        @plsc.parallel_loop(0, CR)
        def _emb(i, _c=c, _eb=eb):
            for d in range(8):
                v = _eb[i, pl.ds(d * 16, 16)]
                if _c == 0:
                    nm_v[i, pl.ds(d * 16, 16)] = v
                else:
                    nm_v[i, pl.ds(d * 16, 16)] += v

        if c + 2 < NCHUNK:
            _fire_emb(c + 2)

    pltpu.sync_copy(nm_v, nm_hbm.at[pl.ds(base_b, BPW)])


_sc_gather = functools.partial(
    pl.kernel,
    out_type=(
        jax.ShapeDtypeStruct((B_C, D_C), jnp.float32),
        jax.ShapeDtypeStruct((B_C, D_C), jnp.float32),
    ),
    mesh=plsc.VectorSubcoreMesh(core_axis_name="c", subcore_axis_name="s"),
    compiler_params=pltpu.CompilerParams(needs_layout_passes=False,
                                         use_tc_tiling_on_sc=False),
    scratch_types=[
        pltpu.VMEM((RPW,), jnp.int32),
        pltpu.VMEM((BPW,), jnp.int32),
        pltpu.VMEM((BPW,), jnp.int32),
        pltpu.VMEM((TPW,), jnp.int32),
        pltpu.VMEM((EPT,), jnp.int32),
        pltpu.VMEM((WPT,), jnp.int32),
        pltpu.VMEM((EPC,), jnp.int32),
        pltpu.VMEM((EPC,), jnp.int32),
        pltpu.VMEM((EPC,), jnp.int32),
        pltpu.VMEM((EPC,), jnp.int32),
        pltpu.VMEM((CR, D_C), jnp.float32),
        pltpu.VMEM((CR, D_C), jnp.float32),
        pltpu.VMEM((BPW, N_TYPES_C), jnp.float32),
        pltpu.VMEM((BPW, D_C), jnp.float32),
        pltpu.VMEM_SHARED((TPW,), jnp.int32),
        pltpu.SemaphoreType.DMA,
        pltpu.SemaphoreType.DMA,
        pltpu.SemaphoreType.DMA,
    ],
)(_sc_body)


def _tc_body(pooled_ref, nm_ref, w1_ref, b1_ref, w2_ref, b2_ref,
             sc_ref, v2_ref):
    v1 = jnp.dot(pooled_ref[:, :N_TYPES_C], w1_ref[...],
                 preferred_element_type=jnp.float32) + b1_ref[...]
    sc_ref[...] = jax.nn.sigmoid(v1)
    p2 = jnp.dot(nm_ref[...], w2_ref[...],
                 preferred_element_type=jnp.float32) + b2_ref[...]
    v2_ref[...] = jnp.concatenate([v1, p2], axis=1)


_tc_heads = pl.pallas_call(
    _tc_body,
    out_shape=(
        jax.ShapeDtypeStruct((B_C, N_TYPES_C), jnp.float32),
        jax.ShapeDtypeStruct((B_C, 2 * N_TYPES_C), jnp.float32),
    ),
)


@jax.jit
def kernel(node_pairs, train_hedges, labels, neighborhedges, hedgetypes,
           nodeEmb, W1, b1, W2, b2):
    npT = node_pairs.astype(jnp.int32).T.reshape(-1)
    th = train_hedges.astype(jnp.int32)
    lb = labels.astype(jnp.int32)
    nbrF = neighborhedges.astype(jnp.int32).T.reshape(-1)
    ht = jnp.concatenate([hedgetypes.astype(jnp.int32),
                          jnp.zeros((HT_PAD - N_HEDGES_C,), jnp.int32)])

    pooled, embsum = _sc_gather(npT, th, lb, nbrF, ht, nodeEmb)

    scores, vector2 = _tc_heads(pooled, embsum, W1, b1.reshape(1, -1),
                                W2 * 0.25, b2.reshape(1, -1))
    return (scores, vector2)


# speedup vs baseline: 1.0214x; 1.0164x over previous
import functools

import jax
import jax.numpy as jnp
from jax import lax
from jax.experimental import pallas as pl
from jax.experimental.pallas import tpu as pltpu
from jax.experimental.pallas import tpu_sc as plsc

N_NODES_C = 100000
N_HEDGES_C = 200000
N_TYPES_C = 16
B_C = 4096
H_C = 4
S_C = 32
D_C = 128

NC = 2
NS = 16
NW = NC * NS
BPW = B_C // NW
RPW = BPW * H_C
NCHUNK = H_C
CR = BPW
EPC = CR * S_C
HT_PAD = 200704
EPT = HT_PAD // NS
WPT = EPT // 8
TPW = HT_PAD // 8


def _sc_body(npT_hbm, th_hbm, lb_hbm, nbrF_hbm, ht_hbm, emb_hbm,
             pooled_hbm, nm_hbm,
             np_v, th_v, lb_v, tp_v, raw_v, pk_v,
             ix0, ix1, nb0, nb1,
             eb0, eb1, hist_v, nm_v, tp_sh,
             sem_ne, sem_emb, sem_tp):
    wid = lax.axis_index("s") * NC + lax.axis_index("c")
    sid = lax.axis_index("s")
    base_b = wid * BPW
    iota16 = lax.iota(jnp.int32, 16)
    ones = jnp.ones((16,), jnp.float32)

    for h in range(H_C):
        pltpu.sync_copy(npT_hbm.at[pl.ds(h * B_C + base_b, BPW)],
                        np_v.at[pl.ds(h * BPW, BPW)])

    tp_cp = pltpu.async_copy(ht_hbm.at[pl.ds(sid * EPT, EPT)], raw_v, sem_tp)

    ixbufs = [ix0, ix1]
    nbufs = [nb0, nb1]
    ebufs = [eb0, eb1]
    ne_cp = [[None] * S_C for _ in range(NCHUNK)]
    emb_cp = [None] * NCHUNK
    soff = iota16 * N_NODES_C

    def _fire_ne(c):
        ix = ixbufs[c % 2]

        @plsc.parallel_loop(0, CR)
        def _mkidx(r):
            n = plsc.load_gather(
                np_v, [jnp.full((16,), c * CR + r, jnp.int32)])
            for k in range(2):
                ix[pl.ds(r * S_C + k * 16, 16)] = n + (soff + k * 16 * N_NODES_C)

        for j in range(S_C):
            ne_cp[c][j] = pltpu.async_copy(
                nbrF_hbm.at[ix.at[pl.ds(j * 128, 128)]],
                nbufs[c % 2].at[pl.ds(j * 128, 128)], sem_ne)

    def _fire_emb(c):
        emb_cp[c] = pltpu.async_copy(
            emb_hbm.at[np_v.at[pl.ds(c * CR, CR)]], ebufs[c % 2], sem_emb)

    pltpu.sync_copy(th_hbm.at[pl.ds(base_b, BPW)], th_v)
    pltpu.sync_copy(lb_hbm.at[pl.ds(base_b, BPW)], lb_v)

    _fire_emb(0)
    _fire_emb(1)
    _fire_ne(0)
    _fire_ne(1)

    tp_cp.wait()

    @plsc.parallel_loop(0, WPT // 16)
    def _pack(i):
        j = (jnp.full((16,), i * 16, jnp.int32) + iota16) * 8
        w = plsc.load_gather(raw_v, [j])
        for k in range(1, 8):
            w = w | lax.shift_left(plsc.load_gather(raw_v, [j + k]), 4 * k)
        pk_v[pl.ds(i * 16, 16)] = w

    pltpu.sync_copy(pk_v, tp_sh.at[pl.ds(sid * WPT, WPT)])

    @plsc.parallel_loop(0, BPW)
    def _zero(i):
        hist_v[i, pl.ds(0, 16)] = jnp.zeros((16,), jnp.float32)

    plsc.subcore_barrier()
    pltpu.sync_copy(tp_sh, tp_v)

    for c in range(NCHUNK):
        for j in range(S_C):
            ne_cp[c][j].wait()
        nb = nbufs[c % 2]

        @plsc.parallel_loop(0, CR)
        def _hist(r, _nb=nb):
            rv = jnp.full((16,), r, jnp.int32)
            thv = plsc.load_gather(th_v, [rv])
            for k in range(2):
                e = _nb[pl.ds(r * S_C + k * 16, 16)]
                w = plsc.load_gather(tp_v, [lax.shift_right_logical(e, 3)])
                sh = lax.shift_left(jnp.bitwise_and(e, 7), 2)
                t = jnp.bitwise_and(lax.shift_right_logical(w, sh), 15)
                m = jnp.not_equal(e, thv)
                plsc.addupdate_scatter(hist_v, [rv, t], ones, mask=m)

        if c + 2 < NCHUNK:
            _fire_ne(c + 2)

    @plsc.parallel_loop(0, BPW)
    def _pooled(b):
        hv = hist_v[b, pl.ds(0, 16)]
        lbl = plsc.load_gather(lb_v, [jnp.full((16,), b, jnp.int32)])
        onehot = jnp.where(iota16 == lbl, 1.0, 0.0).astype(jnp.float32)
        hist_v[b, pl.ds(0, 16)] = hv * (1.0 / 128.0) + onehot

    pltpu.sync_copy(hist_v,
                    pooled_hbm.at[pl.ds(base_b, BPW), pl.ds(0, N_TYPES_C)])

    for c in range(NCHUNK):
        emb_cp[c].wait()
        eb = ebufs[c % 2]

        @plsc.parallel_loop(0, CR)
        def _emb(i, _c=c, _eb=eb):
            for d in range(8):
                v = _eb[i, pl.ds(d * 16, 16)]
                if _c == 0:
                    nm_v[i, pl.ds(d * 16, 16)] = v
                else:
                    nm_v[i, pl.ds(d * 16, 16)] += v

        if c + 2 < NCHUNK:
            _fire_emb(c + 2)

    pltpu.sync_copy(nm_v, nm_hbm.at[pl.ds(base_b, BPW)])


_sc_gather = functools.partial(
    pl.kernel,
    out_type=(
        jax.ShapeDtypeStruct((B_C, D_C), jnp.float32),
        jax.ShapeDtypeStruct((B_C, D_C), jnp.float32),
    ),
    mesh=plsc.VectorSubcoreMesh(core_axis_name="c", subcore_axis_name="s"),
    compiler_params=pltpu.CompilerParams(needs_layout_passes=False,
                                         use_tc_tiling_on_sc=False),
    scratch_types=[
        pltpu.VMEM((RPW,), jnp.int32),
        pltpu.VMEM((BPW,), jnp.int32),
        pltpu.VMEM((BPW,), jnp.int32),
        pltpu.VMEM((TPW,), jnp.int32),
        pltpu.VMEM((EPT,), jnp.int32),
        pltpu.VMEM((WPT,), jnp.int32),
        pltpu.VMEM((EPC,), jnp.int32),
        pltpu.VMEM((EPC,), jnp.int32),
        pltpu.VMEM((EPC,), jnp.int32),
        pltpu.VMEM((EPC,), jnp.int32),
        pltpu.VMEM((CR, D_C), jnp.float32),
        pltpu.VMEM((CR, D_C), jnp.float32),
        pltpu.VMEM((BPW, N_TYPES_C), jnp.float32),
        pltpu.VMEM((BPW, D_C), jnp.float32),
        pltpu.VMEM_SHARED((TPW,), jnp.int32),
        pltpu.SemaphoreType.DMA,
        pltpu.SemaphoreType.DMA,
        pltpu.SemaphoreType.DMA,
    ],
)(_sc_body)


def _tc_body(pooled_ref, nm_ref, w1_ref, b1_ref, w2_ref, b2_ref,
             sc_ref, v2_ref):
    v1 = jnp.dot(pooled_ref[:, :N_TYPES_C], w1_ref[...],
                 preferred_element_type=jnp.float32) + b1_ref[...]
    sc_ref[...] = jax.nn.sigmoid(v1)
    p2 = jnp.dot(nm_ref[...], w2_ref[...],
                 preferred_element_type=jnp.float32) + b2_ref[...]
    v2_ref[...] = jnp.concatenate([v1, p2], axis=1)


_tc_heads = pl.pallas_call(
    _tc_body,
    out_shape=(
        jax.ShapeDtypeStruct((B_C, N_TYPES_C), jnp.float32),
        jax.ShapeDtypeStruct((B_C, 2 * N_TYPES_C), jnp.float32),
    ),
)


@jax.jit
def kernel(node_pairs, train_hedges, labels, neighborhedges, hedgetypes,
           nodeEmb, W1, b1, W2, b2):
    npT = node_pairs.astype(jnp.int32).T.reshape(-1)
    th = train_hedges.astype(jnp.int32)
    lb = labels.astype(jnp.int32)
    nbrF = neighborhedges.astype(jnp.int32).T.reshape(-1)
    ht = jnp.concatenate([hedgetypes.astype(jnp.int32),
                          jnp.zeros((HT_PAD - N_HEDGES_C,), jnp.int32)])

    pooled, embsum = _sc_gather(npT, th, lb, nbrF, ht, nodeEmb)

    scores, vector2 = _tc_heads(pooled, embsum, W1, b1.reshape(1, -1),
                                W2 * 0.25, b2.reshape(1, -1))
    return (scores, vector2)


# speedup vs baseline: 1.0345x; 1.0128x over previous
import functools

import jax
import jax.numpy as jnp
from jax import lax
from jax.experimental import pallas as pl
from jax.experimental.pallas import tpu as pltpu
from jax.experimental.pallas import tpu_sc as plsc

N_NODES_C = 100000
N_HEDGES_C = 200000
N_TYPES_C = 16
B_C = 4096
H_C = 4
S_C = 32
D_C = 128

NC = 2
NS = 16
NW = NC * NS
BPW = B_C // NW
RPW = BPW * H_C
NCHUNK = H_C
CR = BPW
EPC = CR * S_C
EPT = 12544
WPT = EPT // 8
TPW = 25088


def _sc_body(npT_hbm, th_hbm, lb_hbm, nbrF_hbm, ht_hbm, emb_hbm,
             pooled_hbm, nm_hbm,
             np_v, th_v, lb_v, tp_v, raw_v, pk_v,
             ix0, ix1, nb0, nb1,
             eb0, eb1, hist_v, nm_v, tp_sh,
             sem_ne, sem_emb, sem_tp):
    wid = lax.axis_index("s") * NC + lax.axis_index("c")
    sid = lax.axis_index("s")
    base_b = wid * BPW
    iota16 = lax.iota(jnp.int32, 16)
    ones = jnp.ones((16,), jnp.float32)

    for h in range(H_C):
        pltpu.sync_copy(npT_hbm.at[pl.ds(h * B_C + base_b, BPW)],
                        np_v.at[pl.ds(h * BPW, BPW)])

    start_e = pl.multiple_of(jnp.minimum(sid * EPT, N_HEDGES_C - EPT), 64)
    tp_cp = pltpu.async_copy(ht_hbm.at[pl.ds(start_e, EPT)], raw_v, sem_tp)

    ixbufs = [ix0, ix1]
    nbufs = [nb0, nb1]
    ebufs = [eb0, eb1]
    ne_cp = [[None] * S_C for _ in range(NCHUNK)]
    emb_cp = [None] * NCHUNK
    soff = iota16 * N_NODES_C

    def _fire_ne(c):
        ix = ixbufs[c % 2]

        @plsc.parallel_loop(0, CR)
        def _mkidx(r):
            n = plsc.load_gather(
                np_v, [jnp.full((16,), c * CR + r, jnp.int32)])
            for k in range(2):
                ix[pl.ds(r * S_C + k * 16, 16)] = n + (soff + k * 16 * N_NODES_C)

        for j in range(S_C):
            ne_cp[c][j] = pltpu.async_copy(
                nbrF_hbm.at[ix.at[pl.ds(j * 128, 128)]],
                nbufs[c % 2].at[pl.ds(j * 128, 128)], sem_ne)

    def _fire_emb(c):
        emb_cp[c] = pltpu.async_copy(
            emb_hbm.at[np_v.at[pl.ds(c * CR, CR)]], ebufs[c % 2], sem_emb)

    pltpu.sync_copy(th_hbm.at[pl.ds(base_b, BPW)], th_v)
    pltpu.sync_copy(lb_hbm.at[pl.ds(base_b, BPW)], lb_v)

    _fire_emb(0)
    _fire_emb(1)
    _fire_ne(0)
    _fire_ne(1)

    tp_cp.wait()

    @plsc.parallel_loop(0, WPT // 16)
    def _pack(i):
        j = (jnp.full((16,), i * 16, jnp.int32) + iota16) * 8
        w = plsc.load_gather(raw_v, [j])
        for k in range(1, 8):
            w = w | lax.shift_left(plsc.load_gather(raw_v, [j + k]), 4 * k)
        pk_v[pl.ds(i * 16, 16)] = w

    pltpu.sync_copy(
        pk_v,
        tp_sh.at[pl.ds(pl.multiple_of(lax.shift_right_logical(start_e, 3), 8),
                       WPT)])

    @plsc.parallel_loop(0, BPW)
    def _zero(i):
        hist_v[i, pl.ds(0, 16)] = jnp.zeros((16,), jnp.float32)

    plsc.subcore_barrier()
    pltpu.sync_copy(tp_sh, tp_v)

    for c in range(NCHUNK):
        for j in range(S_C):
            ne_cp[c][j].wait()
        nb = nbufs[c % 2]

        @plsc.parallel_loop(0, CR)
        def _hist(r, _nb=nb):
            rv = jnp.full((16,), r, jnp.int32)
            thv = plsc.load_gather(th_v, [rv])
            for k in range(2):
                e = _nb[pl.ds(r * S_C + k * 16, 16)]
                w = plsc.load_gather(tp_v, [lax.shift_right_logical(e, 3)])
                sh = lax.shift_left(jnp.bitwise_and(e, 7), 2)
                t = jnp.bitwise_and(lax.shift_right_logical(w, sh), 15)
                m = jnp.not_equal(e, thv)
                plsc.addupdate_scatter(hist_v, [rv, t], ones, mask=m)

        if c + 2 < NCHUNK:
            _fire_ne(c + 2)

    @plsc.parallel_loop(0, BPW)
    def _pooled(b):
        hv = hist_v[b, pl.ds(0, 16)]
        lbl = plsc.load_gather(lb_v, [jnp.full((16,), b, jnp.int32)])
        onehot = jnp.where(iota16 == lbl, 1.0, 0.0).astype(jnp.float32)
        hist_v[b, pl.ds(0, 16)] = hv * (1.0 / 128.0) + onehot

    pltpu.sync_copy(hist_v,
                    pooled_hbm.at[pl.ds(base_b, BPW), pl.ds(0, N_TYPES_C)])

    for c in range(NCHUNK):
        emb_cp[c].wait()
        eb = ebufs[c % 2]

        @plsc.parallel_loop(0, CR)
        def _emb(i, _c=c, _eb=eb):
            for d in range(8):
                v = _eb[i, pl.ds(d * 16, 16)]
                if _c == 0:
                    nm_v[i, pl.ds(d * 16, 16)] = v
                else:
                    nm_v[i, pl.ds(d * 16, 16)] += v

        if c + 2 < NCHUNK:
            _fire_emb(c + 2)

    pltpu.sync_copy(nm_v, nm_hbm.at[pl.ds(base_b, BPW)])


_sc_gather = functools.partial(
    pl.kernel,
    out_type=(
        jax.ShapeDtypeStruct((B_C, D_C), jnp.float32),
        jax.ShapeDtypeStruct((B_C, D_C), jnp.float32),
    ),
    mesh=plsc.VectorSubcoreMesh(core_axis_name="c", subcore_axis_name="s"),
    compiler_params=pltpu.CompilerParams(needs_layout_passes=False,
                                         use_tc_tiling_on_sc=False),
    scratch_types=[
        pltpu.VMEM((RPW,), jnp.int32),
        pltpu.VMEM((BPW,), jnp.int32),
        pltpu.VMEM((BPW,), jnp.int32),
        pltpu.VMEM((TPW,), jnp.int32),
        pltpu.VMEM((EPT,), jnp.int32),
        pltpu.VMEM((WPT,), jnp.int32),
        pltpu.VMEM((EPC,), jnp.int32),
        pltpu.VMEM((EPC,), jnp.int32),
        pltpu.VMEM((EPC,), jnp.int32),
        pltpu.VMEM((EPC,), jnp.int32),
        pltpu.VMEM((CR, D_C), jnp.float32),
        pltpu.VMEM((CR, D_C), jnp.float32),
        pltpu.VMEM((BPW, N_TYPES_C), jnp.float32),
        pltpu.VMEM((BPW, D_C), jnp.float32),
        pltpu.VMEM_SHARED((TPW,), jnp.int32),
        pltpu.SemaphoreType.DMA,
        pltpu.SemaphoreType.DMA,
        pltpu.SemaphoreType.DMA,
    ],
)(_sc_body)


def _tc_body(pooled_ref, nm_ref, w1_ref, b1_ref, w2_ref, b2_ref,
             sc_ref, v2_ref):
    v1 = jnp.dot(pooled_ref[:, :N_TYPES_C], w1_ref[...],
                 preferred_element_type=jnp.float32) + b1_ref[...]
    sc_ref[...] = jax.nn.sigmoid(v1)
    p2 = jnp.dot(nm_ref[...], w2_ref[...] * 0.25,
                 preferred_element_type=jnp.float32) + b2_ref[...]
    v2_ref[...] = jnp.concatenate([v1, p2], axis=1)


_tc_heads = pl.pallas_call(
    _tc_body,
    out_shape=(
        jax.ShapeDtypeStruct((B_C, N_TYPES_C), jnp.float32),
        jax.ShapeDtypeStruct((B_C, 2 * N_TYPES_C), jnp.float32),
    ),
)


@jax.jit
def kernel(node_pairs, train_hedges, labels, neighborhedges, hedgetypes,
           nodeEmb, W1, b1, W2, b2):
    npT = node_pairs.astype(jnp.int32).T.reshape(-1)
    th = train_hedges.astype(jnp.int32)
    lb = labels.astype(jnp.int32)
    nbrF = neighborhedges.astype(jnp.int32).T.reshape(-1)
    ht = hedgetypes.astype(jnp.int32)

    pooled, embsum = _sc_gather(npT, th, lb, nbrF, ht, nodeEmb)

    scores, vector2 = _tc_heads(pooled, embsum, W1, b1.reshape(1, -1),
                                W2, b2.reshape(1, -1))
    return (scores, vector2)
